# baseline (speedup 1.0000x reference)
import functools

import jax
import jax.numpy as jnp
from jax import lax
from jax.experimental import pallas as pl
from jax.experimental.pallas import tpu as pltpu
from jax.experimental.pallas import tpu_sc as plsc

N_NODES = 10000
N_PAD = 10240
N_EDGES = 160000
IN_CH = 256
CH = 512
CW = 128
NCHUNK = CH // CW
NCORE = 2
NSUB = 16
KB = 80
NB = 63
EPT = NB * KB
E_PAD = NCORE * NSUB * EPT
RPT = N_PAD // NSUB
MB = 256
GRID = N_PAD // MB

_mesh = plsc.VectorSubcoreMesh(core_axis_name="c", subcore_axis_name="s")


@functools.partial(
    pl.kernel,
    mesh=_mesh,
    out_type=jax.ShapeDtypeStruct((NCORE, N_PAD, CW), jnp.float32),
    scratch_types=[
        pltpu.VMEM((NB, KB), jnp.int32),
        pltpu.VMEM((KB, CW), jnp.float32),
        pltpu.VMEM_SHARED((N_PAD, CW), jnp.float32),
        pltpu.SemaphoreType.DMA,
    ],
)
def _deg_sc(dst_hbm, ones_hbm, zeros_hbm, out_hbm, dst_v, ones_v, acc, sem):
    c = lax.axis_index("c")
    s = lax.axis_index("s")
    pltpu.sync_copy(dst_hbm.at[c, s], dst_v)
    pltpu.sync_copy(ones_hbm, ones_v)
    row0 = s * RPT
    pltpu.sync_copy(zeros_hbm, acc.at[pl.ds(row0, RPT)])
    plsc.subcore_barrier()

    def body(j, carry):
        pltpu.async_copy(ones_v, acc.at[dst_v.at[j]], sem, add=True)
        return carry

    lax.fori_loop(0, NB, body, 0)

    def drain(j, carry):
        pltpu.make_async_copy(ones_v, acc.at[dst_v.at[j]], sem).wait()
        return carry

    lax.fori_loop(0, NB, drain, 0)
    plsc.subcore_barrier()
    pltpu.sync_copy(acc.at[pl.ds(row0, RPT)],
                    out_hbm.at[c, pl.ds(row0, RPT)])


@functools.partial(
    pl.kernel,
    mesh=_mesh,
    out_type=[jax.ShapeDtypeStruct((NCORE, N_PAD, CW), jnp.float32)] * NCHUNK,
    scratch_types=[
        pltpu.VMEM((NB, KB), jnp.int32),
        pltpu.VMEM((NB, KB), jnp.int32),
        pltpu.VMEM((3, KB, CW), jnp.float32),
        pltpu.VMEM_SHARED((N_PAD, CW), jnp.float32),
        pltpu.SemaphoreType.DMA,
        pltpu.SemaphoreType.DMA,
        pltpu.SemaphoreType.DMA,
    ],
)
def _agg_sc(g0, g1, g2, g3, src_hbm, dst_hbm, zeros_hbm,
            p0, p1, p2, p3, src_v, dst_v, rows_v, acc, sem0, sem1, sem2):
    c = lax.axis_index("c")
    s = lax.axis_index("s")
    pltpu.sync_copy(src_hbm.at[c, s], src_v)
    pltpu.sync_copy(dst_hbm.at[c, s], dst_v)
    gs = (g0, g1, g2, g3)
    ps = (p0, p1, p2, p3)
    gsems = (sem0, sem1, sem2)
    row0 = s * RPT
    for cc in range(NCHUNK):
        pltpu.sync_copy(zeros_hbm, acc.at[pl.ds(row0, RPT)])
        plsc.subcore_barrier()
        g_cc = gs[cc]

        def gcp(j, buf, g_cc=g_cc):
            return pltpu.make_async_copy(
                g_cc.at[src_v.at[j]], rows_v.at[buf], gsems[buf])

        gcp(0, 0).start()
        gcp(1, 1).start()
        gcp(2, 2).start()

        def body(i, carry, gcp=gcp):
            for k in range(3):
                j = 3 * i + k
                gcp(j, k).wait()
                pltpu.sync_copy(rows_v.at[k], acc.at[dst_v.at[j]], add=True)

                @pl.when(j + 3 < NB)
                def _(j=j, k=k):
                    gcp(j + 3, k).start()

            return carry

        lax.fori_loop(0, NB // 3, body, 0)
        plsc.subcore_barrier()
        pltpu.sync_copy(acc.at[pl.ds(row0, RPT)],
                        ps[cc].at[c, pl.ds(row0, RPT)])


def _dinv_of(dp_ref):
    deg = dp_ref[0, :, 0:1] + dp_ref[1, :, 0:1] + 1.0
    return lax.rsqrt(jnp.maximum(deg, 1.0))


def _t0_body(x_ref, w_ref, dp_ref, *out_refs):
    dinv = _dinv_of(dp_ref)
    g = jnp.dot(x_ref[...], w_ref[...],
                preferred_element_type=jnp.float32) * dinv
    for cc in range(NCHUNK):
        out_refs[cc][...] = g[:, cc * CW:(cc + 1) * CW]


_t0 = pl.pallas_call(
    _t0_body,
    grid=(GRID,),
    in_specs=[
        pl.BlockSpec((MB, IN_CH), lambda i: (i, 0)),
        pl.BlockSpec((IN_CH, CH), lambda i: (0, 0)),
        pl.BlockSpec((NCORE, MB, CW), lambda i: (0, i, 0)),
    ],
    out_specs=[pl.BlockSpec((MB, CW), lambda i: (i, 0))] * NCHUNK,
    out_shape=[jax.ShapeDtypeStruct((N_PAD, CW), jnp.float32)] * NCHUNK,
)


def _mid_body(p0, p1, p2, p3, g0, g1, g2, g3, dp_ref, w_ref, b_ref,
              *out_refs):
    dinv = _dinv_of(dp_ref)
    ps = (p0, p1, p2, p3)
    gs = (g0, g1, g2, g3)
    acc = jnp.zeros((MB, CH), jnp.float32)
    for cc in range(NCHUNK):
        y = jnp.maximum(
            dinv * (ps[cc][0] + ps[cc][1] + gs[cc][...])
            + b_ref[0, cc * CW:(cc + 1) * CW], 0.0)
        acc = acc + jnp.dot(y, w_ref[cc * CW:(cc + 1) * CW, :],
                            preferred_element_type=jnp.float32)
    gout = acc * dinv
    for cc in range(NCHUNK):
        out_refs[cc][...] = gout[:, cc * CW:(cc + 1) * CW]


_mid = pl.pallas_call(
    _mid_body,
    grid=(GRID,),
    in_specs=(
        [pl.BlockSpec((NCORE, MB, CW), lambda i: (0, i, 0))] * NCHUNK
        + [pl.BlockSpec((MB, CW), lambda i: (i, 0))] * NCHUNK
        + [
            pl.BlockSpec((NCORE, MB, CW), lambda i: (0, i, 0)),
            pl.BlockSpec((CH, CH), lambda i: (0, 0)),
            pl.BlockSpec((1, CH), lambda i: (0, 0)),
        ]
    ),
    out_specs=[pl.BlockSpec((MB, CW), lambda i: (i, 0))] * NCHUNK,
    out_shape=[jax.ShapeDtypeStruct((N_PAD, CW), jnp.float32)] * NCHUNK,
)


def _fin_body(p0, p1, p2, p3, g0, g1, g2, g3, dp_ref, b_ref, out_ref):
    dinv = _dinv_of(dp_ref)
    ps = (p0, p1, p2, p3)
    gs = (g0, g1, g2, g3)
    for cc in range(NCHUNK):
        y = jnp.maximum(
            dinv * (ps[cc][0] + ps[cc][1] + gs[cc][...])
            + b_ref[0, cc * CW:(cc + 1) * CW], 0.0)
        out_ref[:, cc * CW:(cc + 1) * CW] = y


_fin = pl.pallas_call(
    _fin_body,
    grid=(GRID,),
    in_specs=(
        [pl.BlockSpec((NCORE, MB, CW), lambda i: (0, i, 0))] * NCHUNK
        + [pl.BlockSpec((MB, CW), lambda i: (i, 0))] * NCHUNK
        + [
            pl.BlockSpec((NCORE, MB, CW), lambda i: (0, i, 0)),
            pl.BlockSpec((1, CH), lambda i: (0, 0)),
        ]
    ),
    out_specs=pl.BlockSpec((MB, CH), lambda i: (i, 0)),
    out_shape=jax.ShapeDtypeStruct((N_PAD, CH), jnp.float32),
)


def kernel(x, edge_index, W1, b1, W2, b2, W3, b3):
    epad = N_NODES + (jnp.arange(E_PAD - N_EDGES, dtype=jnp.int32)
                      % (N_PAD - N_NODES))
    src = jnp.concatenate([edge_index[0].astype(jnp.int32), epad])
    src = src.reshape(NCORE, NSUB, NB, KB)
    dst = jnp.concatenate([edge_index[1].astype(jnp.int32), epad])
    dst = dst.reshape(NCORE, NSUB, NB, KB)
    xp = jnp.pad(x, ((0, N_PAD - N_NODES), (0, 0)))
    ones_r = jnp.ones((KB, CW), jnp.float32)
    zrow = jnp.zeros((RPT, CW), jnp.float32)
    b1r = b1.reshape(1, CH)
    b2r = b2.reshape(1, CH)
    b3r = b3.reshape(1, CH)

    dp = _deg_sc(dst, ones_r, zrow)
    g = _t0(xp, W1, dp)
    p = _agg_sc(*g, src, dst, zrow)
    g = _mid(*p, *g, dp, W2, b1r)
    p = _agg_sc(*g, src, dst, zrow)
    g = _mid(*p, *g, dp, W3, b2r)
    p = _agg_sc(*g, src, dst, zrow)
    out = _fin(*p, *g, dp, b3r)
    return out[:N_NODES]


# speedup vs baseline: 1.0187x; 1.0187x over previous
import functools

import jax
import jax.numpy as jnp
from jax import lax
from jax.experimental import pallas as pl
from jax.experimental.pallas import tpu as pltpu
from jax.experimental.pallas import tpu_sc as plsc

N_NODES = 10000
N_PAD = 10240
N_EDGES = 160000
IN_CH = 256
CH = 512
CW = 128
NCHUNK = CH // CW
NCORE = 2
NSUB = 16
KB = 80
NB = 63
EPT = NB * KB
E_PAD = NCORE * NSUB * EPT
RPT = N_PAD // NSUB
MB = 256
GRID = N_PAD // MB

_mesh = plsc.VectorSubcoreMesh(core_axis_name="c", subcore_axis_name="s")


@functools.partial(
    pl.kernel,
    mesh=_mesh,
    out_type=jax.ShapeDtypeStruct((NCORE, N_PAD, CW), jnp.float32),
    scratch_types=[
        pltpu.VMEM((NB, KB), jnp.int32),
        pltpu.VMEM((KB, CW), jnp.float32),
        pltpu.VMEM_SHARED((N_PAD, CW), jnp.float32),
        pltpu.SemaphoreType.DMA,
    ],
)
def _deg_sc(dst_hbm, ones_hbm, zeros_hbm, out_hbm, dst_v, ones_v, acc, sem):
    c = lax.axis_index("c")
    s = lax.axis_index("s")
    pltpu.sync_copy(dst_hbm.at[c, s], dst_v)
    pltpu.sync_copy(ones_hbm, ones_v)
    row0 = s * RPT
    pltpu.sync_copy(zeros_hbm, acc.at[pl.ds(row0, RPT)])
    plsc.subcore_barrier()

    def body(j, carry):
        pltpu.async_copy(ones_v, acc.at[dst_v.at[j]], sem, add=True)
        return carry

    lax.fori_loop(0, NB, body, 0)

    def drain(j, carry):
        pltpu.make_async_copy(ones_v, acc.at[dst_v.at[j]], sem).wait()
        return carry

    lax.fori_loop(0, NB, drain, 0)
    plsc.subcore_barrier()
    pltpu.sync_copy(acc.at[pl.ds(row0, RPT)],
                    out_hbm.at[c, pl.ds(row0, RPT)])


@functools.partial(
    pl.kernel,
    mesh=_mesh,
    out_type=[jax.ShapeDtypeStruct((NCORE, N_PAD, CW), jnp.float32)] * NCHUNK,
    scratch_types=[
        pltpu.VMEM((NB, KB), jnp.int32),
        pltpu.VMEM((NB, KB), jnp.int32),
        pltpu.VMEM((3, KB, CW), jnp.float32),
        pltpu.VMEM_SHARED((N_PAD, CW), jnp.float32),
        pltpu.SemaphoreType.DMA,
        pltpu.SemaphoreType.DMA,
        pltpu.SemaphoreType.DMA,
    ],
)
def _agg_sc(g0, g1, g2, g3, src_hbm, dst_hbm, zeros_hbm,
            p0, p1, p2, p3, src_v, dst_v, rows_v, acc, sem0, sem1, sem2):
    c = lax.axis_index("c")
    s = lax.axis_index("s")
    pltpu.sync_copy(src_hbm.at[c, s], src_v)
    pltpu.sync_copy(dst_hbm.at[c, s], dst_v)
    gs = (g0, g1, g2, g3)
    ps = (p0, p1, p2, p3)
    gsems = (sem0, sem1, sem2)
    row0 = s * RPT

    def gcp(j, buf, cc):
        return pltpu.make_async_copy(
            gs[cc].at[src_v.at[j]], rows_v.at[buf], gsems[buf])

    def prime(cc):
        gcp(0, 0, cc).start()
        gcp(1, 1, cc).start()
        gcp(2, 2, cc).start()

    pltpu.sync_copy(zeros_hbm, acc.at[pl.ds(row0, RPT)])
    plsc.subcore_barrier()
    prime(0)
    for cc in range(NCHUNK):

        def body(i, carry, cc=cc):
            for k in range(3):
                j = 3 * i + k
                gcp(j, k, cc).wait()
                pltpu.sync_copy(rows_v.at[k], acc.at[dst_v.at[j]], add=True)

                @pl.when(j + 3 < NB)
                def _(j=j, k=k, cc=cc):
                    gcp(j + 3, k, cc).start()

            return carry

        lax.fori_loop(0, NB // 3, body, 0)
        if cc + 1 < NCHUNK:
            prime(cc + 1)
        plsc.subcore_barrier()
        pltpu.sync_copy(acc.at[pl.ds(row0, RPT)],
                        ps[cc].at[c, pl.ds(row0, RPT)])
        if cc + 1 < NCHUNK:
            pltpu.sync_copy(zeros_hbm, acc.at[pl.ds(row0, RPT)])
            plsc.subcore_barrier()


def _dinv_of(dp_ref):
    deg = dp_ref[0, :, 0:1] + dp_ref[1, :, 0:1] + 1.0
    return lax.rsqrt(jnp.maximum(deg, 1.0))


def _t0_body(x_ref, w_ref, dp_ref, *out_refs):
    dinv = _dinv_of(dp_ref)
    g = jnp.dot(x_ref[...], w_ref[...],
                preferred_element_type=jnp.float32) * dinv
    for cc in range(NCHUNK):
        out_refs[cc][...] = g[:, cc * CW:(cc + 1) * CW]


_t0 = pl.pallas_call(
    _t0_body,
    grid=(GRID,),
    in_specs=[
        pl.BlockSpec((MB, IN_CH), lambda i: (i, 0)),
        pl.BlockSpec((IN_CH, CH), lambda i: (0, 0)),
        pl.BlockSpec((NCORE, MB, CW), lambda i: (0, i, 0)),
    ],
    out_specs=[pl.BlockSpec((MB, CW), lambda i: (i, 0))] * NCHUNK,
    out_shape=[jax.ShapeDtypeStruct((N_PAD, CW), jnp.float32)] * NCHUNK,
)


def _mid_body(p0, p1, p2, p3, g0, g1, g2, g3, dp_ref, w_ref, b_ref,
              *out_refs):
    dinv = _dinv_of(dp_ref)
    ps = (p0, p1, p2, p3)
    gs = (g0, g1, g2, g3)
    acc = jnp.zeros((MB, CH), jnp.float32)
    for cc in range(NCHUNK):
        y = jnp.maximum(
            dinv * (ps[cc][0] + ps[cc][1] + gs[cc][...])
            + b_ref[0, cc * CW:(cc + 1) * CW], 0.0)
        acc = acc + jnp.dot(y, w_ref[cc * CW:(cc + 1) * CW, :],
                            preferred_element_type=jnp.float32)
    gout = acc * dinv
    for cc in range(NCHUNK):
        out_refs[cc][...] = gout[:, cc * CW:(cc + 1) * CW]


_mid = pl.pallas_call(
    _mid_body,
    grid=(GRID,),
    in_specs=(
        [pl.BlockSpec((NCORE, MB, CW), lambda i: (0, i, 0))] * NCHUNK
        + [pl.BlockSpec((MB, CW), lambda i: (i, 0))] * NCHUNK
        + [
            pl.BlockSpec((NCORE, MB, CW), lambda i: (0, i, 0)),
            pl.BlockSpec((CH, CH), lambda i: (0, 0)),
            pl.BlockSpec((1, CH), lambda i: (0, 0)),
        ]
    ),
    out_specs=[pl.BlockSpec((MB, CW), lambda i: (i, 0))] * NCHUNK,
    out_shape=[jax.ShapeDtypeStruct((N_PAD, CW), jnp.float32)] * NCHUNK,
)


def _fin_body(p0, p1, p2, p3, g0, g1, g2, g3, dp_ref, b_ref, out_ref):
    dinv = _dinv_of(dp_ref)
    ps = (p0, p1, p2, p3)
    gs = (g0, g1, g2, g3)
    for cc in range(NCHUNK):
        y = jnp.maximum(
            dinv * (ps[cc][0] + ps[cc][1] + gs[cc][...])
            + b_ref[0, cc * CW:(cc + 1) * CW], 0.0)
        out_ref[:, cc * CW:(cc + 1) * CW] = y


_fin = pl.pallas_call(
    _fin_body,
    grid=(GRID,),
    in_specs=(
        [pl.BlockSpec((NCORE, MB, CW), lambda i: (0, i, 0))] * NCHUNK
        + [pl.BlockSpec((MB, CW), lambda i: (i, 0))] * NCHUNK
        + [
            pl.BlockSpec((NCORE, MB, CW), lambda i: (0, i, 0)),
            pl.BlockSpec((1, CH), lambda i: (0, 0)),
        ]
    ),
    out_specs=pl.BlockSpec((MB, CH), lambda i: (i, 0)),
    out_shape=jax.ShapeDtypeStruct((N_PAD, CH), jnp.float32),
)


def kernel(x, edge_index, W1, b1, W2, b2, W3, b3):
    epad = N_NODES + (jnp.arange(E_PAD - N_EDGES, dtype=jnp.int32)
                      % (N_PAD - N_NODES))
    src = jnp.concatenate([edge_index[0].astype(jnp.int32), epad])
    src = src.reshape(NCORE, NSUB, NB, KB)
    dst = jnp.concatenate([edge_index[1].astype(jnp.int32), epad])
    dst = dst.reshape(NCORE, NSUB, NB, KB)
    xp = jnp.pad(x, ((0, N_PAD - N_NODES), (0, 0)))
    ones_r = jnp.ones((KB, CW), jnp.float32)
    zrow = jnp.zeros((RPT, CW), jnp.float32)
    b1r = b1.reshape(1, CH)
    b2r = b2.reshape(1, CH)
    b3r = b3.reshape(1, CH)

    dp = _deg_sc(dst, ones_r, zrow)
    g = _t0(xp, W1, dp)
    p = _agg_sc(*g, src, dst, zrow)
    g = _mid(*p, *g, dp, W2, b1r)
    p = _agg_sc(*g, src, dst, zrow)
    g = _mid(*p, *g, dp, W3, b2r)
    p = _agg_sc(*g, src, dst, zrow)
    out = _fin(*p, *g, dp, b3r)
    return out[:N_NODES]
